# Initial kernel scaffold; baseline (speedup 1.0000x reference)
import jax
import jax.numpy as jnp
from jax.experimental import pallas as pl


def kernel(x, W):
    raise NotImplementedError("write your pallas kernel here")


# speedup vs baseline: 1.3422x; 1.3422x over previous
import functools

import jax
import jax.numpy as jnp
from jax.experimental import pallas as pl
from jax.experimental.pallas import tpu as pltpu

N_EXP = 8
BLK_T = 1024


def _gate_kernel(x_ref, w_ref, wout_ref, iout_ref):
    x = x_ref[...]
    w = w_ref[...]
    s = jax.lax.dot_general(
        x, w, (((1,), (1,)), ((), ())), preferred_element_type=jnp.float32
    )
    col = jax.lax.broadcasted_iota(jnp.int32, s.shape, 1)

    m1 = jnp.max(s, axis=1, keepdims=True)
    denom = jnp.sum(jnp.exp(s - m1), axis=1, keepdims=True)
    i1 = jnp.min(jnp.where(s == m1, col, N_EXP), axis=1, keepdims=True)
    s2 = jnp.where(col == i1, -jnp.inf, s)
    m2 = jnp.max(s2, axis=1, keepdims=True)
    i2 = jnp.min(jnp.where(s2 == m2, col, N_EXP), axis=1, keepdims=True)

    inv = 1.0 / denom
    w1 = inv
    w2 = jnp.exp(m2 - m1) * inv
    wout_ref[...] = jnp.concatenate([w1, w2], axis=1)
    iout_ref[...] = jnp.concatenate([i1, i2], axis=1)


@jax.jit
def kernel(x, W):
    n_tokens, dim = x.shape
    grid = (n_tokens // BLK_T,)
    wout, iout = pl.pallas_call(
        _gate_kernel,
        grid=grid,
        in_specs=[
            pl.BlockSpec((BLK_T, dim), lambda i: (i, 0)),
            pl.BlockSpec((N_EXP, dim), lambda i: (0, 0)),
        ],
        out_specs=[
            pl.BlockSpec((BLK_T, 2), lambda i: (i, 0)),
            pl.BlockSpec((BLK_T, 2), lambda i: (i, 0)),
        ],
        out_shape=[
            jax.ShapeDtypeStruct((n_tokens, 2), jnp.float32),
            jax.ShapeDtypeStruct((n_tokens, 2), jnp.int32),
        ],
    )(x, W)
    return wout, iout


# speedup vs baseline: 1.4294x; 1.0649x over previous
import functools

import jax
import jax.numpy as jnp
from jax.experimental import pallas as pl
from jax.experimental.pallas import tpu as pltpu

N_EXP = 8
BLK_T = 2048


def _gate_kernel(x_ref, w_ref, wout_ref, iout_ref):
    x = x_ref[...]
    w = w_ref[...]
    s = jax.lax.dot_general(
        x, w, (((1,), (1,)), ((), ())), preferred_element_type=jnp.float32
    )
    col = jax.lax.broadcasted_iota(jnp.int32, s.shape, 1)

    m1 = jnp.max(s, axis=1, keepdims=True)
    denom = jnp.sum(jnp.exp(s - m1), axis=1, keepdims=True)
    i1 = jnp.min(jnp.where(s == m1, col, N_EXP), axis=1, keepdims=True)
    s2 = jnp.where(col == i1, -jnp.inf, s)
    m2 = jnp.max(s2, axis=1, keepdims=True)
    i2 = jnp.min(jnp.where(s2 == m2, col, N_EXP), axis=1, keepdims=True)

    inv = 1.0 / denom
    w1 = inv
    w2 = jnp.exp(m2 - m1) * inv
    wout_ref[...] = jnp.concatenate([w1, w2], axis=1)
    iout_ref[...] = jnp.concatenate([i1, i2], axis=1)


@jax.jit
def kernel(x, W):
    n_tokens, dim = x.shape
    grid = (n_tokens // BLK_T,)
    wout, iout = pl.pallas_call(
        _gate_kernel,
        grid=grid,
        in_specs=[
            pl.BlockSpec((BLK_T, dim), lambda i: (i, 0)),
            pl.BlockSpec((N_EXP, dim), lambda i: (0, 0)),
        ],
        out_specs=[
            pl.BlockSpec((BLK_T, 2), lambda i: (i, 0)),
            pl.BlockSpec((BLK_T, 2), lambda i: (i, 0)),
        ],
        out_shape=[
            jax.ShapeDtypeStruct((n_tokens, 2), jnp.float32),
            jax.ShapeDtypeStruct((n_tokens, 2), jnp.int32),
        ],
    )(x, W)
    return wout, iout
